# baseline (speedup 1.0000x reference)
import functools

import jax
import jax.numpy as jnp
from jax import lax
from jax.experimental import pallas as pl
from jax.experimental.pallas import tpu as pltpu
from jax.experimental.pallas import tpu_sc as plsc

N = 4096
E = 262144
H = 128
NC = 2
NS = 16
L = 16
NW = NC * NS
EPW = E // NW
CHUNK = 128
ROWS = EPW // CHUNK


SLICE = N // NS


def _sc_pass_body(with_deg, *refs):
    if with_deg:
        (vals_hbm, edges_hbm, sum_out, deg_out,
         x_v, src_v, dst_v, acc_v, deg_v, red_v, res_v, stage_sh,
         dstage_sh, sem) = refs
    else:
        (vals_hbm, edges_hbm, sum_out,
         x_v, src_v, dst_v, acc_v, red_v, res_v, stage_sh, sem) = refs

    cid = lax.axis_index("c")
    sid = lax.axis_index("s")
    wid = cid * NS + sid
    f32 = jnp.float32

    with jax.named_scope("sc_stage"):
        stage = [
            pltpu.async_copy(vals_hbm.at[0], x_v, sem),
            pltpu.async_copy(edges_hbm.at[pl.ds(wid * EPW, EPW)], src_v, sem),
            pltpu.async_copy(edges_hbm.at[pl.ds(E + wid * EPW, EPW)],
                             dst_v, sem),
        ]

    with jax.named_scope("sc_zero"):
        def zb(i, c):
            acc_v[pl.ds(i * L, L)] = jnp.zeros((L,), f32)
            if with_deg:
                deg_v[pl.ds(i * L, L)] = jnp.zeros((L,), f32)
            return c
        lax.fori_loop(0, N // L, zb, 0)
        for d in stage:
            d.wait()

    ones16 = jnp.ones((L,), f32)

    with jax.named_scope("sc_edges"):
        def gbody(j, c):
            for k in range(CHUNK // L):
                sidx = src_v[pl.ds(j * CHUNK + k * L, L)]
                didx = dst_v[pl.ds(j * CHUNK + k * L, L)]
                v16 = plsc.load_gather(x_v, [sidx])
                plsc.addupdate_scatter(acc_v, [didx], v16)
                if with_deg:
                    plsc.addupdate_scatter(deg_v, [didx], ones16)
            return c
        lax.fori_loop(0, ROWS, gbody, 0)

    with jax.named_scope("sc_publish"):
        pltpu.sync_copy(acc_v, stage_sh.at[pl.ds(sid * N, N)])
        if with_deg:
            pltpu.sync_copy(deg_v, dstage_sh.at[pl.ds(sid * N, N)])
        plsc.subcore_barrier()

    stages = [(stage_sh, sum_out)]
    if with_deg:
        stages.append((dstage_sh, deg_out))
    with jax.named_scope("sc_reduce"):
      for sh, out in stages:
        descs = [pltpu.async_copy(
            sh.at[pl.ds(r * N + sid * SLICE, SLICE)],
            red_v.at[pl.ds(r * SLICE, SLICE)], sem)
            for r in range(NS)]
        for d in descs:
            d.wait()

        def rbody(k, c):
            acc16 = red_v[pl.ds(k * L, L)]
            for r in range(1, NS):
                acc16 = acc16 + red_v[pl.ds(r * SLICE + k * L, L)]
            res_v[pl.ds(k * L, L)] = acc16
            return c
        lax.fori_loop(0, SLICE // L, rbody, 0)

        pltpu.sync_copy(res_v, out.at[cid, pl.ds(sid * SLICE, SLICE)])


@functools.cache
def _make_sc_pass(with_deg):
    mesh = plsc.VectorSubcoreMesh(core_axis_name="c", subcore_axis_name="s")
    f32 = jnp.float32
    nacc = 2 if with_deg else 1
    out_type = [jax.ShapeDtypeStruct((NC, N), f32)]
    scratch = [
        pltpu.VMEM((N,), f32),
        pltpu.VMEM((EPW,), jnp.int32),
        pltpu.VMEM((EPW,), jnp.int32),
        pltpu.VMEM((N,), f32),
    ]
    if with_deg:
        out_type = out_type + [jax.ShapeDtypeStruct((NC, N), f32)]
        scratch = scratch + [pltpu.VMEM((N,), f32)]
    scratch = scratch + [
        pltpu.VMEM((NS * SLICE,), f32),
        pltpu.VMEM((SLICE,), f32),
        pltpu.VMEM_SHARED((NS * N,), f32),
    ]
    if with_deg:
        scratch = scratch + [pltpu.VMEM_SHARED((NS * N,), f32)]
    scratch = scratch + [pltpu.SemaphoreType.DMA]
    return pl.kernel(
        functools.partial(_sc_pass_body, with_deg),
        mesh=mesh,
        out_type=out_type,
        scratch_types=scratch,
        compiler_params=pltpu.CompilerParams(needs_layout_passes=False),
    )


def _tc_mid_body(x_ref, sx_ref, dg_ref, ws1_ref, wn1_ref, b1_ref,
                 ws2_ref, wn2_ref, s_ref, t_ref, d_ref):
    x = x_ref[...]
    deg = dg_ref[0:1, :] + dg_ref[1:2, :]
    sx = sx_ref[0:1, :] + sx_ref[1:2, :]
    agg = sx / jnp.maximum(deg, 1.0)
    h1 = jnp.tanh(ws1_ref[...] * x + wn1_ref[...] * agg + b1_ref[...])
    t_ref[...] = lax.dot_general(
        ws2_ref[...], h1, (((1,), (0,)), ((), ())),
        preferred_element_type=jnp.float32)
    s_ref[...] = lax.dot_general(
        wn2_ref[...], h1, (((1,), (0,)), ((), ())),
        preferred_element_type=jnp.float32)
    d_ref[...] = deg


_tc_mid = pl.pallas_call(
    _tc_mid_body,
    out_shape=[jax.ShapeDtypeStruct((1, N), jnp.float32)] * 3,
)


RB = 512
NBLK = N // RB


NBH = NBLK // 2


@functools.cache
def _make_tc_bot(c):
    def body(x_ref, pin_ref, w3_ref, p_ref, h_ref):
        i = pl.program_id(0)

        @pl.when(i == 0)
        def _init():
            h_ref[...] = jnp.tanh(x_ref[...])
            p_ref[...] = pin_ref[...]

        hs = h_ref[0:1, pl.ds((c * NBH + i) * RB, RB)]
        p_ref[...] += lax.dot_general(
            hs, w3_ref[...], (((1,), (0,)), ((), ())),
            preferred_element_type=jnp.float32)

    return pl.pallas_call(
        body,
        grid=(NBH,),
        in_specs=[
            pl.BlockSpec((1, N), lambda i: (0, 0)),
            pl.BlockSpec((1, N), lambda i: (0, 0)),
            pl.BlockSpec((RB, N), lambda i: (NBLK + c * NBH + i, 0)),
        ],
        out_specs=pl.BlockSpec((1, N), lambda i: (0, 0)),
        out_shape=jax.ShapeDtypeStruct((1, N), jnp.float32),
        scratch_shapes=[pltpu.VMEM((1, N), jnp.float32)],
    )


def _tc_top_body(t_ref, ss_ref, dg_ref, b2_ref, p_ref, w3_ref,
                 o1_ref, o2_ref, h_ref):
    i = pl.program_id(0)

    @pl.when(i == 0)
    def _init():
        deg = dg_ref[...]
        agg2 = (ss_ref[0:1, :] + ss_ref[1:2, :]) / jnp.maximum(deg, 1.0)
        o1 = t_ref[...] + agg2 + b2_ref[0, 0]
        o1_ref[...] = o1
        h_ref[...] = jnp.tanh(o1)
        o2_ref[...] = p_ref[...]

    hs = h_ref[0:1, pl.ds(i * RB, RB)]
    o2_ref[...] += lax.dot_general(
        hs, w3_ref[...], (((1,), (0,)), ((), ())),
        preferred_element_type=jnp.float32)


_tc_top = pl.pallas_call(
    _tc_top_body,
    grid=(NBLK,),
    in_specs=[
        pl.BlockSpec((1, N), lambda i: (0, 0)),
        pl.BlockSpec((NC, N), lambda i: (0, 0)),
        pl.BlockSpec((1, N), lambda i: (0, 0)),
        pl.BlockSpec((1, 1), lambda i: (0, 0)),
        pl.BlockSpec((1, N), lambda i: (0, 0)),
        pl.BlockSpec((RB, N), lambda i: (i, 0)),
    ],
    out_specs=[
        pl.BlockSpec((1, N), lambda i: (0, 0)),
        pl.BlockSpec((1, N), lambda i: (0, 0)),
    ],
    out_shape=[jax.ShapeDtypeStruct((1, N), jnp.float32)] * 2,
    scratch_shapes=[pltpu.VMEM((1, N), jnp.float32)],
)


def kernel(node_features, edge_index, W_self1, W_neigh1, b1,
           W_self2, W_neigh2, b2, W3, b3):
    x_row = node_features.reshape(1, N)
    e_flat = edge_index.reshape(2 * E)

    sumx_p, deg_p = _make_sc_pass(True)(x_row, e_flat)

    p1 = _make_tc_bot(0)(x_row, b3.reshape(1, N), W3)
    sumx_p, deg_p, p1 = lax.optimization_barrier((sumx_p, deg_p, p1))

    s_row, t_row, deg_row = _tc_mid(
        x_row, sumx_p, deg_p,
        W_self1.reshape(H, 1), W_neigh1.reshape(H, 1), b1.reshape(H, 1),
        W_self2.reshape(1, H), W_neigh2.reshape(1, H))

    (sums_p,) = _make_sc_pass(False)(s_row, e_flat)

    p2 = _make_tc_bot(1)(x_row, p1, W3)
    sums_p, p2 = lax.optimization_barrier((sums_p, p2))

    o1, o2 = _tc_top(t_row, sums_p, deg_row, b2.reshape(1, 1),
                     p2, W3)
    return o1, o2


# speedup vs baseline: 1.0081x; 1.0081x over previous
import functools

import jax
import jax.numpy as jnp
from jax import lax
from jax.experimental import pallas as pl
from jax.experimental.pallas import tpu as pltpu
from jax.experimental.pallas import tpu_sc as plsc

N = 4096
E = 262144
H = 128
NC = 2
NS = 16
L = 16
NW = NC * NS
EPW = E // NW
CHUNK = 128
ROWS = EPW // CHUNK


SLICE = N // NS


def _sc_pass_body(with_deg, *refs):
    if with_deg:
        (vals_hbm, edges_hbm, sum_out, deg_out,
         x_v, src_v, dst_v, acc_v, deg_v, red_v, res_v, stage_sh,
         dstage_sh, sem) = refs
    else:
        (vals_hbm, edges_hbm, sum_out,
         x_v, src_v, dst_v, acc_v, red_v, res_v, stage_sh, sem) = refs

    cid = lax.axis_index("c")
    sid = lax.axis_index("s")
    wid = cid * NS + sid
    f32 = jnp.float32

    with jax.named_scope("sc_stage"):
        stage = [
            pltpu.async_copy(vals_hbm.at[0], x_v, sem),
            pltpu.async_copy(edges_hbm.at[0, pl.ds(wid * EPW, EPW)],
                             src_v, sem),
            pltpu.async_copy(edges_hbm.at[1, pl.ds(wid * EPW, EPW)],
                             dst_v, sem),
        ]

    with jax.named_scope("sc_zero"):
        def zb(i, c):
            acc_v[pl.ds(i * L, L)] = jnp.zeros((L,), f32)
            if with_deg:
                deg_v[pl.ds(i * L, L)] = jnp.zeros((L,), f32)
            return c
        lax.fori_loop(0, N // L, zb, 0)
        for d in stage:
            d.wait()

    ones16 = jnp.ones((L,), f32)

    with jax.named_scope("sc_edges"):
        def gbody(j, c):
            for k in range(CHUNK // L):
                sidx = src_v[pl.ds(j * CHUNK + k * L, L)]
                didx = dst_v[pl.ds(j * CHUNK + k * L, L)]
                v16 = plsc.load_gather(x_v, [sidx])
                plsc.addupdate_scatter(acc_v, [didx], v16)
                if with_deg:
                    plsc.addupdate_scatter(deg_v, [didx], ones16)
            return c
        lax.fori_loop(0, ROWS, gbody, 0)

    with jax.named_scope("sc_publish"):
        pltpu.sync_copy(acc_v, stage_sh.at[pl.ds(sid * N, N)])
        if with_deg:
            pltpu.sync_copy(deg_v, dstage_sh.at[pl.ds(sid * N, N)])
        plsc.subcore_barrier()

    stages = [(stage_sh, sum_out)]
    if with_deg:
        stages.append((dstage_sh, deg_out))
    with jax.named_scope("sc_reduce"):
      for sh, out in stages:
        descs = [pltpu.async_copy(
            sh.at[pl.ds(r * N + sid * SLICE, SLICE)],
            red_v.at[pl.ds(r * SLICE, SLICE)], sem)
            for r in range(NS)]
        for d in descs:
            d.wait()

        def rbody(k, c):
            acc16 = red_v[pl.ds(k * L, L)]
            for r in range(1, NS):
                acc16 = acc16 + red_v[pl.ds(r * SLICE + k * L, L)]
            res_v[pl.ds(k * L, L)] = acc16
            return c
        lax.fori_loop(0, SLICE // L, rbody, 0)

        pltpu.sync_copy(res_v, out.at[cid, pl.ds(sid * SLICE, SLICE)])


@functools.cache
def _make_sc_pass(with_deg):
    mesh = plsc.VectorSubcoreMesh(core_axis_name="c", subcore_axis_name="s")
    f32 = jnp.float32
    nacc = 2 if with_deg else 1
    out_type = [jax.ShapeDtypeStruct((NC, N), f32)]
    scratch = [
        pltpu.VMEM((N,), f32),
        pltpu.VMEM((EPW,), jnp.int32),
        pltpu.VMEM((EPW,), jnp.int32),
        pltpu.VMEM((N,), f32),
    ]
    if with_deg:
        out_type = out_type + [jax.ShapeDtypeStruct((NC, N), f32)]
        scratch = scratch + [pltpu.VMEM((N,), f32)]
    scratch = scratch + [
        pltpu.VMEM((NS * SLICE,), f32),
        pltpu.VMEM((SLICE,), f32),
        pltpu.VMEM_SHARED((NS * N,), f32),
    ]
    if with_deg:
        scratch = scratch + [pltpu.VMEM_SHARED((NS * N,), f32)]
    scratch = scratch + [pltpu.SemaphoreType.DMA]
    return pl.kernel(
        functools.partial(_sc_pass_body, with_deg),
        mesh=mesh,
        out_type=out_type,
        scratch_types=scratch,
        compiler_params=pltpu.CompilerParams(needs_layout_passes=False),
    )


def _tc_mid_body(x_ref, sx_ref, dg_ref, ws1_ref, wn1_ref, b1_ref,
                 ws2_ref, wn2_ref, s_ref, t_ref, d_ref):
    x = x_ref[...]
    deg = dg_ref[0:1, :] + dg_ref[1:2, :]
    sx = sx_ref[0:1, :] + sx_ref[1:2, :]
    agg = sx / jnp.maximum(deg, 1.0)
    h1 = jnp.tanh(ws1_ref[...] * x + wn1_ref[...] * agg + b1_ref[...])
    t_ref[...] = lax.dot_general(
        ws2_ref[...], h1, (((1,), (0,)), ((), ())),
        preferred_element_type=jnp.float32)
    s_ref[...] = lax.dot_general(
        wn2_ref[...], h1, (((1,), (0,)), ((), ())),
        preferred_element_type=jnp.float32)
    d_ref[...] = deg


_tc_mid = pl.pallas_call(
    _tc_mid_body,
    out_shape=[jax.ShapeDtypeStruct((1, N), jnp.float32)] * 3,
)


RB = 512
NBLK = N // RB


NBH = NBLK // 2


@functools.cache
def _make_tc_bot(c):
    def body(x_ref, pin_ref, w3_ref, p_ref, h_ref):
        i = pl.program_id(0)

        @pl.when(i == 0)
        def _init():
            h_ref[...] = jnp.tanh(x_ref[...])
            p_ref[...] = pin_ref[...]

        hs = h_ref[0:1, pl.ds((c * NBH + i) * RB, RB)]
        p_ref[...] += lax.dot_general(
            hs, w3_ref[...], (((1,), (0,)), ((), ())),
            preferred_element_type=jnp.float32)

    return pl.pallas_call(
        body,
        grid=(NBH,),
        in_specs=[
            pl.BlockSpec((1, N), lambda i: (0, 0)),
            pl.BlockSpec((1, N), lambda i: (0, 0)),
            pl.BlockSpec((RB, N), lambda i: (NBLK + c * NBH + i, 0)),
        ],
        out_specs=pl.BlockSpec((1, N), lambda i: (0, 0)),
        out_shape=jax.ShapeDtypeStruct((1, N), jnp.float32),
        scratch_shapes=[pltpu.VMEM((1, N), jnp.float32)],
    )


def _tc_top_body(t_ref, ss_ref, dg_ref, b2_ref, p_ref, w3_ref,
                 o1_ref, o2_ref, h_ref):
    i = pl.program_id(0)

    @pl.when(i == 0)
    def _init():
        deg = dg_ref[...]
        agg2 = (ss_ref[0:1, :] + ss_ref[1:2, :]) / jnp.maximum(deg, 1.0)
        o1 = t_ref[...] + agg2 + b2_ref[0, 0]
        o1_ref[...] = o1
        h_ref[...] = jnp.tanh(o1)
        o2_ref[...] = p_ref[...]

    hs = h_ref[0:1, pl.ds(i * RB, RB)]
    o2_ref[...] += lax.dot_general(
        hs, w3_ref[...], (((1,), (0,)), ((), ())),
        preferred_element_type=jnp.float32)


_tc_top = pl.pallas_call(
    _tc_top_body,
    grid=(NBLK,),
    in_specs=[
        pl.BlockSpec((1, N), lambda i: (0, 0)),
        pl.BlockSpec((NC, N), lambda i: (0, 0)),
        pl.BlockSpec((1, N), lambda i: (0, 0)),
        pl.BlockSpec((1, 1), lambda i: (0, 0)),
        pl.BlockSpec((1, N), lambda i: (0, 0)),
        pl.BlockSpec((RB, N), lambda i: (i, 0)),
    ],
    out_specs=[
        pl.BlockSpec((1, N), lambda i: (0, 0)),
        pl.BlockSpec((1, N), lambda i: (0, 0)),
    ],
    out_shape=[jax.ShapeDtypeStruct((1, N), jnp.float32)] * 2,
    scratch_shapes=[pltpu.VMEM((1, N), jnp.float32)],
)


def kernel(node_features, edge_index, W_self1, W_neigh1, b1,
           W_self2, W_neigh2, b2, W3, b3):
    x_row = node_features.reshape(1, N)

    sumx_p, deg_p = _make_sc_pass(True)(x_row, edge_index)

    p1 = _make_tc_bot(0)(x_row, b3.reshape(1, N), W3)
    sumx_p, deg_p, p1 = lax.optimization_barrier((sumx_p, deg_p, p1))

    s_row, t_row, deg_row = _tc_mid(
        x_row, sumx_p, deg_p,
        W_self1.reshape(H, 1), W_neigh1.reshape(H, 1), b1.reshape(H, 1),
        W_self2.reshape(1, H), W_neigh2.reshape(1, H))

    (sums_p,) = _make_sc_pass(False)(s_row, edge_index)

    p2 = _make_tc_bot(1)(x_row, p1, W3)
    sums_p, p2 = lax.optimization_barrier((sums_p, p2))

    o1, o2 = _tc_top(t_row, sums_p, deg_row, b2.reshape(1, 1),
                     p2, W3)
    return o1, o2


# speedup vs baseline: 1.0094x; 1.0013x over previous
import functools

import jax
import jax.numpy as jnp
from jax import lax
from jax.experimental import pallas as pl
from jax.experimental.pallas import tpu as pltpu
from jax.experimental.pallas import tpu_sc as plsc

N = 4096
E = 262144
H = 128
NC = 2
NS = 16
L = 16
NW = NC * NS
EPW = E // NW
CHUNK = 128
ROWS = EPW // CHUNK


SLICE = N // NS


def _sc_pass_body(with_deg, *refs):
    if with_deg:
        (vals_hbm, edges_hbm, sum_out, deg_out,
         x_v, src_v, dst_v, vals_v, acc_v, deg_v, red_v, res_v, stage_sh,
         dstage_sh, sem) = refs
    else:
        (vals_hbm, edges_hbm, sum_out,
         x_v, src_v, dst_v, vals_v, acc_v, red_v, res_v, stage_sh, sem) = refs

    cid = lax.axis_index("c")
    sid = lax.axis_index("s")
    wid = cid * NS + sid
    f32 = jnp.float32

    with jax.named_scope("sc_stage"):
        stage = [
            pltpu.async_copy(vals_hbm.at[0], x_v, sem),
            pltpu.async_copy(edges_hbm.at[0, pl.ds(wid * EPW, EPW)],
                             src_v, sem),
            pltpu.async_copy(edges_hbm.at[1, pl.ds(wid * EPW, EPW)],
                             dst_v, sem),
        ]

    with jax.named_scope("sc_zero"):
        def zb(i, c):
            acc_v[pl.ds(i * L, L)] = jnp.zeros((L,), f32)
            if with_deg:
                deg_v[pl.ds(i * L, L)] = jnp.zeros((L,), f32)
            return c
        lax.fori_loop(0, N // L, zb, 0)
        for d in stage:
            d.wait()

    ones16 = jnp.ones((L,), f32)

    with jax.named_scope("sc_gather"):
        @plsc.parallel_loop(0, EPW // L, unroll=4)
        def _gather(i):
            sidx = src_v[pl.ds(i * L, L)]
            vals_v[pl.ds(i * L, L)] = plsc.load_gather(x_v, [sidx])

    with jax.named_scope("sc_scatter"):
        def sbody(j, c):
            for k in range(CHUNK // L):
                o = j * CHUNK + k * L
                didx = dst_v[pl.ds(o, L)]
                plsc.addupdate_scatter(acc_v, [didx], vals_v[pl.ds(o, L)])
                if with_deg:
                    plsc.addupdate_scatter(deg_v, [didx], ones16)
            return c
        lax.fori_loop(0, ROWS, sbody, 0)

    with jax.named_scope("sc_publish"):
        pltpu.sync_copy(acc_v, stage_sh.at[pl.ds(sid * N, N)])
        if with_deg:
            pltpu.sync_copy(deg_v, dstage_sh.at[pl.ds(sid * N, N)])
        plsc.subcore_barrier()

    stages = [(stage_sh, sum_out)]
    if with_deg:
        stages.append((dstage_sh, deg_out))
    with jax.named_scope("sc_reduce"):
      for sh, out in stages:
        descs = [pltpu.async_copy(
            sh.at[pl.ds(r * N + sid * SLICE, SLICE)],
            red_v.at[pl.ds(r * SLICE, SLICE)], sem)
            for r in range(NS)]
        for d in descs:
            d.wait()

        def rbody(k, c):
            acc16 = red_v[pl.ds(k * L, L)]
            for r in range(1, NS):
                acc16 = acc16 + red_v[pl.ds(r * SLICE + k * L, L)]
            res_v[pl.ds(k * L, L)] = acc16
            return c
        lax.fori_loop(0, SLICE // L, rbody, 0)

        pltpu.sync_copy(res_v, out.at[cid, pl.ds(sid * SLICE, SLICE)])


@functools.cache
def _make_sc_pass(with_deg):
    mesh = plsc.VectorSubcoreMesh(core_axis_name="c", subcore_axis_name="s")
    f32 = jnp.float32
    nacc = 2 if with_deg else 1
    out_type = [jax.ShapeDtypeStruct((NC, N), f32)]
    scratch = [
        pltpu.VMEM((N,), f32),
        pltpu.VMEM((EPW,), jnp.int32),
        pltpu.VMEM((EPW,), jnp.int32),
        pltpu.VMEM((EPW,), f32),
        pltpu.VMEM((N,), f32),
    ]
    if with_deg:
        out_type = out_type + [jax.ShapeDtypeStruct((NC, N), f32)]
        scratch = scratch + [pltpu.VMEM((N,), f32)]
    scratch = scratch + [
        pltpu.VMEM((NS * SLICE,), f32),
        pltpu.VMEM((SLICE,), f32),
        pltpu.VMEM_SHARED((NS * N,), f32),
    ]
    if with_deg:
        scratch = scratch + [pltpu.VMEM_SHARED((NS * N,), f32)]
    scratch = scratch + [pltpu.SemaphoreType.DMA]
    return pl.kernel(
        functools.partial(_sc_pass_body, with_deg),
        mesh=mesh,
        out_type=out_type,
        scratch_types=scratch,
        compiler_params=pltpu.CompilerParams(needs_layout_passes=False),
    )


def _tc_mid_body(x_ref, sx_ref, dg_ref, ws1_ref, wn1_ref, b1_ref,
                 ws2_ref, wn2_ref, s_ref, t_ref, d_ref):
    x = x_ref[...]
    deg = dg_ref[0:1, :] + dg_ref[1:2, :]
    sx = sx_ref[0:1, :] + sx_ref[1:2, :]
    agg = sx / jnp.maximum(deg, 1.0)
    h1 = jnp.tanh(ws1_ref[...] * x + wn1_ref[...] * agg + b1_ref[...])
    t_ref[...] = lax.dot_general(
        ws2_ref[...], h1, (((1,), (0,)), ((), ())),
        preferred_element_type=jnp.float32)
    s_ref[...] = lax.dot_general(
        wn2_ref[...], h1, (((1,), (0,)), ((), ())),
        preferred_element_type=jnp.float32)
    d_ref[...] = deg


_tc_mid = pl.pallas_call(
    _tc_mid_body,
    out_shape=[jax.ShapeDtypeStruct((1, N), jnp.float32)] * 3,
)


RB = 512
NBLK = N // RB


NBH = NBLK // 2


@functools.cache
def _make_tc_bot(c):
    def body(x_ref, pin_ref, w3_ref, p_ref, h_ref):
        i = pl.program_id(0)

        @pl.when(i == 0)
        def _init():
            h_ref[...] = jnp.tanh(x_ref[...])
            p_ref[...] = pin_ref[...]

        hs = h_ref[0:1, pl.ds((c * NBH + i) * RB, RB)]
        p_ref[...] += lax.dot_general(
            hs, w3_ref[...], (((1,), (0,)), ((), ())),
            preferred_element_type=jnp.float32)

    return pl.pallas_call(
        body,
        grid=(NBH,),
        in_specs=[
            pl.BlockSpec((1, N), lambda i: (0, 0)),
            pl.BlockSpec((1, N), lambda i: (0, 0)),
            pl.BlockSpec((RB, N), lambda i: (NBLK + c * NBH + i, 0)),
        ],
        out_specs=pl.BlockSpec((1, N), lambda i: (0, 0)),
        out_shape=jax.ShapeDtypeStruct((1, N), jnp.float32),
        scratch_shapes=[pltpu.VMEM((1, N), jnp.float32)],
    )


def _tc_top_body(t_ref, ss_ref, dg_ref, b2_ref, p_ref, w3_ref,
                 o1_ref, o2_ref, h_ref):
    i = pl.program_id(0)

    @pl.when(i == 0)
    def _init():
        deg = dg_ref[...]
        agg2 = (ss_ref[0:1, :] + ss_ref[1:2, :]) / jnp.maximum(deg, 1.0)
        o1 = t_ref[...] + agg2 + b2_ref[0, 0]
        o1_ref[...] = o1
        h_ref[...] = jnp.tanh(o1)
        o2_ref[...] = p_ref[...]

    hs = h_ref[0:1, pl.ds(i * RB, RB)]
    o2_ref[...] += lax.dot_general(
        hs, w3_ref[...], (((1,), (0,)), ((), ())),
        preferred_element_type=jnp.float32)


_tc_top = pl.pallas_call(
    _tc_top_body,
    grid=(NBLK,),
    in_specs=[
        pl.BlockSpec((1, N), lambda i: (0, 0)),
        pl.BlockSpec((NC, N), lambda i: (0, 0)),
        pl.BlockSpec((1, N), lambda i: (0, 0)),
        pl.BlockSpec((1, 1), lambda i: (0, 0)),
        pl.BlockSpec((1, N), lambda i: (0, 0)),
        pl.BlockSpec((RB, N), lambda i: (i, 0)),
    ],
    out_specs=[
        pl.BlockSpec((1, N), lambda i: (0, 0)),
        pl.BlockSpec((1, N), lambda i: (0, 0)),
    ],
    out_shape=[jax.ShapeDtypeStruct((1, N), jnp.float32)] * 2,
    scratch_shapes=[pltpu.VMEM((1, N), jnp.float32)],
)


def kernel(node_features, edge_index, W_self1, W_neigh1, b1,
           W_self2, W_neigh2, b2, W3, b3):
    x_row = node_features.reshape(1, N)

    sumx_p, deg_p = _make_sc_pass(True)(x_row, edge_index)

    p1 = _make_tc_bot(0)(x_row, b3.reshape(1, N), W3)
    sumx_p, deg_p, p1 = lax.optimization_barrier((sumx_p, deg_p, p1))

    s_row, t_row, deg_row = _tc_mid(
        x_row, sumx_p, deg_p,
        W_self1.reshape(H, 1), W_neigh1.reshape(H, 1), b1.reshape(H, 1),
        W_self2.reshape(1, H), W_neigh2.reshape(1, H))

    (sums_p,) = _make_sc_pass(False)(s_row, edge_index)

    p2 = _make_tc_bot(1)(x_row, p1, W3)
    sums_p, p2 = lax.optimization_barrier((sums_p, p2))

    o1, o2 = _tc_top(t_row, sums_p, deg_row, b2.reshape(1, 1),
                     p2, W3)
    return o1, o2
